# baseline (speedup 1.0000x reference)
import functools

import jax
import jax.numpy as jnp
from jax import lax
from jax.experimental import pallas as pl
from jax.experimental.pallas import tpu as pltpu
from jax.experimental.pallas import tpu_sc as plsc

VOCAB = 1000000
SEQ = 200
HID = 64
BATCH = 4096
EPS = 1e-12

NW = 32
BPW = BATCH // NW
RPW = BPW * SEQ
SPC = 2
SUP = SPC * BPW
NSUP = SEQ // SPC
GROUPS = SUP // 16


def _rsqrt(v):
    i = plsc.bitcast(v, jnp.int32)
    i = jnp.int32(0x5F3759DF) - (i >> 1)
    y = plsc.bitcast(i, jnp.float32)
    for _ in range(3):
        y = y * (1.5 - 0.5 * v * y * y)
    return y


def _make_emb_kernel():
    mesh = plsc.VectorSubcoreMesh(core_axis_name="c", subcore_axis_name="s")

    @functools.partial(
        pl.kernel,
        mesh=mesh,
        compiler_params=pltpu.CompilerParams(
            needs_layout_passes=False, use_tc_tiling_on_sc=True),
        out_type=jax.ShapeDtypeStruct((SEQ, 8, NW, 8, 128), jnp.float32),
        scratch_types=[
            pltpu.VMEM((SUP, 128), jnp.float32),
            pltpu.VMEM((SUP, 128), jnp.float32),
            pltpu.VMEM((SPC, 8, 8, 128), jnp.float32),
            pltpu.VMEM((SPC, 8, 8, 128), jnp.float32),
            pltpu.VMEM((SUP,), jnp.int32),
            pltpu.VMEM((SUP,), jnp.int32),
            pltpu.VMEM((SUP,), jnp.int32),
            pltpu.VMEM((SUP,), jnp.int32),
            pltpu.VMEM((HID * 16,), jnp.float32),
            pltpu.VMEM((SEQ * HID,), jnp.float32),
            pltpu.VMEM((HID * 16,), jnp.float32),
            pltpu.VMEM((HID * 16,), jnp.float32),
            pltpu.SemaphoreType.DMA,
            pltpu.SemaphoreType.DMA,
            pltpu.SemaphoreType.DMA,
            pltpu.SemaphoreType.DMA,
        ],
    )
    def emb(state_hbm, table_hbm, pos_hbm, gamma_hbm, beta_hbm, out_hbm,
            rows0, rows1, y0, y1, ib0, ib1, ih0, ih1,
            xpack, pos_v, gamma_v, beta_v, gs0, gs1, os0, os1):
        rows = (rows0, rows1)
        ybuf = (y0, y1)
        ibuf = (ib0, ib1)
        hbuf = (ih0, ih1)
        gsem = (gs0, gs1)
        osem = (os0, os1)
        wid = lax.axis_index("s") * 2 + lax.axis_index("c")
        pltpu.sync_copy(pos_hbm, pos_v)
        pltpu.sync_copy(gamma_hbm, gamma_v)
        pltpu.sync_copy(beta_hbm, beta_v)
        lanes = lax.iota(jnp.int32, 16)

        def fire_gather(c, b):
            pltpu.sync_copy(state_hbm.at[wid, pl.ds(c * SUP, SUP)], ibuf[b])
            for m in range(SUP // 16):
                iv = ibuf[b][pl.ds(m * 16, 16)]
                hbuf[b][pl.ds(m * 16, 16)] = iv >> 1
            for j in range(SPC):
                pltpu.async_copy(
                    table_hbm.at[hbuf[b].at[pl.ds(j * 128, 128)]],
                    rows[b].at[pl.ds(j * 128, 128)], gsem[b])

        def wait_gather(b):
            pltpu.make_async_copy(
                table_hbm.at[pl.ds(0, SUP)], rows[b], gsem[b]).wait()

        def fire_out(c, b):
            pltpu.async_copy(
                ybuf[b], out_hbm.at[pl.ds(c * SPC, SPC), :, wid], osem[b])

        def wait_out(b):
            pltpu.make_async_copy(
                ybuf[b], out_hbm.at[pl.ds(0, SPC), :, wid], osem[b]).wait()

        def compute(c, b):
            buf = rows[b]

            def group_body(gi, _):
                s_i = gi // 8
                b0 = (gi % 8) * 16
                rr = s_i * 128 + b0 + lanes
                pbase = (c * SPC + s_i) * HID
                par = plsc.load_gather(ibuf[b], [rr]) & 1
                par64 = par << 6

                hvec = lanes
                sa = jnp.zeros((16,), jnp.float32)
                sb = jnp.zeros((16,), jnp.float32)
                s2a = jnp.zeros((16,), jnp.float32)
                s2b = jnp.zeros((16,), jnp.float32)
                for k in range(HID):
                    t = plsc.load_gather(buf, [rr, par64 + hvec])
                    p = plsc.load_gather(pos_v, [pbase + hvec])
                    x = t + p
                    plsc.store_scatter(xpack, [k * 16 + lanes], x)
                    if k % 2 == 0:
                        sa = sa + x
                        s2a = s2a + x * x
                    else:
                        sb = sb + x
                        s2b = s2b + x * x
                    if k < HID - 1:
                        hvec = (hvec + 1) & (HID - 1)
                mean = (sa + sb) * (1.0 / HID)
                var = (s2a + s2b) * (1.0 / HID) - mean * mean
                rstd = _rsqrt(var + EPS)

                avec = (-lanes) & (HID - 1)
                si_v = jnp.full((16,), s_i, jnp.int32)
                blv = b0 + lanes
                for h in range(HID):
                    xh = plsc.load_gather(xpack, [(avec << 4) + lanes])
                    gam = gamma_v[pl.ds(h * 16, 16)]
                    bet = beta_v[pl.ds(h * 16, 16)]
                    yv = (xh - mean) * rstd * gam + bet
                    plsc.store_scatter(
                        ybuf[b],
                        [si_v, jnp.full((16,), h // 8, jnp.int32),
                         jnp.full((16,), h % 8, jnp.int32), blv], yv)
                    if h < HID - 1:
                        avec = (avec + 1) & (HID - 1)
                return 0

            lax.fori_loop(0, GROUPS, group_body, 0)

        def body(c, b):
            wait_gather(b)

            @pl.when(c >= 2)
            def _():
                wait_out(b)

            @pl.when(c + 1 < NSUP)
            def _():
                fire_gather(c + 1, 1 - b)

            compute(c, b)
            fire_out(c, b)

        fire_gather(0, 0)

        def loop_body(c, _):
            @pl.when(c % 2 == 0)
            def _():
                body(c, 0)

            @pl.when(c % 2 == 1)
            def _():
                body(c, 1)
            return 0

        lax.fori_loop(0, NSUP, loop_body, 0)
        wait_out(0)
        wait_out(1)

    return emb


_emb_kernel = _make_emb_kernel()


def kernel(state, token_table, pos_table, ln_gamma, ln_beta):
    table2 = token_table.reshape(VOCAB // 2, 128)
    state_t = state.reshape(NW, BPW, SEQ).transpose(0, 2, 1).reshape(NW, RPW)
    pos_flat = pos_table.reshape(-1)
    gamma_splat = jnp.repeat(ln_gamma, 16)
    beta_splat = jnp.repeat(ln_beta, 16)
    out5 = _emb_kernel(state_t, table2, pos_flat, gamma_splat, beta_splat)
    return out5.transpose(2, 4, 0, 1, 3).reshape(BATCH, SEQ, HID)


# speedup vs baseline: 1.7992x; 1.7992x over previous
import functools

import jax
import jax.numpy as jnp
from jax import lax
from jax.experimental import pallas as pl
from jax.experimental.pallas import tpu as pltpu
from jax.experimental.pallas import tpu_sc as plsc

VOCAB = 1000000
SEQ = 200
HID = 64
BATCH = 4096
EPS = 1e-12

NW = 32
BPW = BATCH // NW
RPW = BPW * SEQ
SPC = 2
SUP = SPC * BPW
NSUP = SEQ // SPC
GROUPS = SUP // 16


def _rsqrt(v):
    i = plsc.bitcast(v, jnp.int32)
    i = jnp.int32(0x5F3759DF) - (i >> 1)
    y = plsc.bitcast(i, jnp.float32)
    for _ in range(3):
        y = y * (1.5 - 0.5 * v * y * y)
    return y


def _make_emb_kernel():
    mesh = plsc.VectorSubcoreMesh(core_axis_name="c", subcore_axis_name="s")

    @functools.partial(
        pl.kernel,
        mesh=mesh,
        compiler_params=pltpu.CompilerParams(
            needs_layout_passes=False, use_tc_tiling_on_sc=True),
        out_type=jax.ShapeDtypeStruct((SEQ, 8, NW, 8, 128), jnp.float32),
        scratch_types=[
            pltpu.VMEM((SUP, 128), jnp.float32),
            pltpu.VMEM((SUP, 128), jnp.float32),
            pltpu.VMEM((SPC, 8, 8, 128), jnp.float32),
            pltpu.VMEM((SPC, 8, 8, 128), jnp.float32),
            pltpu.VMEM((SUP,), jnp.int32),
            pltpu.VMEM((SUP,), jnp.int32),
            pltpu.VMEM((SUP,), jnp.int32),
            pltpu.VMEM((SUP,), jnp.int32),
            pltpu.VMEM((HID * 16,), jnp.float32),
            pltpu.VMEM((SEQ * HID,), jnp.float32),
            pltpu.VMEM((HID * 16,), jnp.float32),
            pltpu.VMEM((HID * 16,), jnp.float32),
            pltpu.SemaphoreType.DMA,
            pltpu.SemaphoreType.DMA,
            pltpu.SemaphoreType.DMA,
            pltpu.SemaphoreType.DMA,
        ],
    )
    def emb(state_hbm, table_hbm, pos_hbm, gamma_hbm, beta_hbm, out_hbm,
            rows0, rows1, y0, y1, ib0, ib1, ih0, ih1,
            xpack, pos_v, gamma_v, beta_v, gs0, gs1, os0, os1):
        rows = (rows0, rows1)
        ybuf = (y0, y1)
        ibuf = (ib0, ib1)
        hbuf = (ih0, ih1)
        gsem = (gs0, gs1)
        osem = (os0, os1)
        wid = lax.axis_index("s") * 2 + lax.axis_index("c")
        pltpu.sync_copy(pos_hbm, pos_v)
        pltpu.sync_copy(gamma_hbm, gamma_v)
        pltpu.sync_copy(beta_hbm, beta_v)
        lanes = lax.iota(jnp.int32, 16)

        def fire_gather(c, b):
            pltpu.sync_copy(state_hbm.at[wid, pl.ds(c * SUP, SUP)], ibuf[b])
            for m in range(SUP // 16):
                iv = ibuf[b][pl.ds(m * 16, 16)]
                hbuf[b][pl.ds(m * 16, 16)] = iv >> 1
            for j in range(SPC):
                pltpu.async_copy(
                    table_hbm.at[hbuf[b].at[pl.ds(j * 128, 128)]],
                    rows[b].at[pl.ds(j * 128, 128)], gsem[b])

        def wait_gather(b):
            pltpu.make_async_copy(
                table_hbm.at[pl.ds(0, SUP)], rows[b], gsem[b]).wait()

        def fire_out(c, b):
            pltpu.async_copy(
                ybuf[b], out_hbm.at[pl.ds(c * SPC, SPC), :, wid], osem[b])

        def wait_out(b):
            pltpu.make_async_copy(
                ybuf[b], out_hbm.at[pl.ds(0, SPC), :, wid], osem[b]).wait()

        def compute(c, b):
            buf = rows[b]

            def group_body(gi, _):
                s_i = gi // 8
                b0 = (gi % 8) * 16
                rr = s_i * 128 + b0 + lanes
                pbase = (c * SPC + s_i) * HID
                par = plsc.load_gather(ibuf[b], [rr]) & 1
                par64 = par << 6

                zero = jnp.zeros((16,), jnp.float32)

                @plsc.parallel_loop(0, HID, 1, unroll=8,
                                    carry=(zero, zero))
                def _p1(k, carry):
                    s, s2 = carry
                    hvec = (lanes + k) & (HID - 1)
                    t = plsc.load_gather(buf, [rr, par64 + hvec])
                    p = plsc.load_gather(pos_v, [pbase + hvec])
                    x = t + p
                    plsc.store_scatter(xpack, [k * 16 + lanes], x)
                    return s + x, s2 + x * x

                s, s2 = _p1
                mean = s * (1.0 / HID)
                var = s2 * (1.0 / HID) - mean * mean
                rstd = _rsqrt(var + EPS)

                si_v = jnp.full((16,), s_i, jnp.int32)
                blv = b0 + lanes

                @plsc.parallel_loop(0, HID, 1, unroll=8)
                def _p2(h):
                    avec = (h - lanes) & (HID - 1)
                    xh = plsc.load_gather(xpack, [(avec << 4) + lanes])
                    gam = gamma_v[pl.ds(h * 16, 16)]
                    bet = beta_v[pl.ds(h * 16, 16)]
                    yv = (xh - mean) * rstd * gam + bet
                    plsc.store_scatter(
                        ybuf[b],
                        [si_v, jnp.full((16,), h >> 3, jnp.int32),
                         jnp.full((16,), h & 7, jnp.int32), blv], yv)
                return 0

            lax.fori_loop(0, GROUPS, group_body, 0)

        def body(c, b):
            wait_gather(b)

            @pl.when(c >= 2)
            def _():
                wait_out(b)

            @pl.when(c + 1 < NSUP)
            def _():
                fire_gather(c + 1, 1 - b)

            compute(c, b)
            fire_out(c, b)

        fire_gather(0, 0)

        def loop_body(c, _):
            @pl.when(c % 2 == 0)
            def _():
                body(c, 0)

            @pl.when(c % 2 == 1)
            def _():
                body(c, 1)
            return 0

        lax.fori_loop(0, NSUP, loop_body, 0)
        wait_out(0)
        wait_out(1)

    return emb


_emb_kernel = _make_emb_kernel()


def kernel(state, token_table, pos_table, ln_gamma, ln_beta):
    table2 = token_table.reshape(VOCAB // 2, 128)
    state_t = state.reshape(NW, BPW, SEQ).transpose(0, 2, 1).reshape(NW, RPW)
    pos_flat = pos_table.reshape(-1)
    gamma_splat = jnp.repeat(ln_gamma, 16)
    beta_splat = jnp.repeat(ln_beta, 16)
    out5 = _emb_kernel(state_t, table2, pos_flat, gamma_splat, beta_splat)
    return out5.transpose(2, 4, 0, 1, 3).reshape(BATCH, SEQ, HID)


# speedup vs baseline: 1.8353x; 1.0201x over previous
import functools

import jax
import jax.numpy as jnp
from jax import lax
from jax.experimental import pallas as pl
from jax.experimental.pallas import tpu as pltpu
from jax.experimental.pallas import tpu_sc as plsc

VOCAB = 1000000
SEQ = 200
HID = 64
BATCH = 4096
EPS = 1e-12

NW = 32
BPW = BATCH // NW
RPW = BPW * SEQ
SPC = 2
SUP = SPC * BPW
NSUP = SEQ // SPC
GROUPS = SUP // 16


def _rsqrt(v):
    i = plsc.bitcast(v, jnp.int32)
    i = jnp.int32(0x5F3759DF) - (i >> 1)
    y = plsc.bitcast(i, jnp.float32)
    for _ in range(3):
        y = y * (1.5 - 0.5 * v * y * y)
    return y


def _make_emb_kernel():
    mesh = plsc.VectorSubcoreMesh(core_axis_name="c", subcore_axis_name="s")

    @functools.partial(
        pl.kernel,
        mesh=mesh,
        compiler_params=pltpu.CompilerParams(
            needs_layout_passes=False, use_tc_tiling_on_sc=True),
        out_type=jax.ShapeDtypeStruct((SEQ, 8, NW, 8, 128), jnp.float32),
        scratch_types=[
            pltpu.VMEM((SUP, 128), jnp.float32),
            pltpu.VMEM((SUP, 128), jnp.float32),
            pltpu.VMEM((SPC, 8, 8, 128), jnp.float32),
            pltpu.VMEM((SPC, 8, 8, 128), jnp.float32),
            pltpu.VMEM((SUP,), jnp.int32),
            pltpu.VMEM((SUP,), jnp.int32),
            pltpu.VMEM((SUP,), jnp.int32),
            pltpu.VMEM((SUP,), jnp.int32),
            pltpu.VMEM((HID * 16,), jnp.float32),
            pltpu.VMEM((SEQ * HID,), jnp.float32),
            pltpu.VMEM((HID * 16,), jnp.float32),
            pltpu.VMEM((HID * 16,), jnp.float32),
            pltpu.SemaphoreType.DMA,
            pltpu.SemaphoreType.DMA,
            pltpu.SemaphoreType.DMA,
            pltpu.SemaphoreType.DMA,
        ],
    )
    def emb(state_hbm, table_hbm, pos_hbm, gamma_hbm, beta_hbm, out_hbm,
            rows0, rows1, y0, y1, ib0, ib1, ih0, ih1,
            xpack, pos_v, gamma_v, beta_v, gs0, gs1, os0, os1):
        rows = (rows0, rows1)
        ybuf = (y0, y1)
        ibuf = (ib0, ib1)
        hbuf = (ih0, ih1)
        gsem = (gs0, gs1)
        osem = (os0, os1)
        wid = lax.axis_index("s") * 2 + lax.axis_index("c")
        pltpu.sync_copy(pos_hbm, pos_v)
        pltpu.sync_copy(gamma_hbm, gamma_v)
        pltpu.sync_copy(beta_hbm, beta_v)
        lanes = lax.iota(jnp.int32, 16)

        def fire_gather(c, b):
            pltpu.sync_copy(state_hbm.at[wid, pl.ds(c * SUP, SUP)], ibuf[b])
            for m in range(SUP // 16):
                iv = ibuf[b][pl.ds(m * 16, 16)]
                hbuf[b][pl.ds(m * 16, 16)] = iv >> 1
            for j in range(SPC):
                pltpu.async_copy(
                    table_hbm.at[hbuf[b].at[pl.ds(j * 128, 128)]],
                    rows[b].at[pl.ds(j * 128, 128)], gsem[b])

        def wait_gather(b):
            pltpu.make_async_copy(
                table_hbm.at[pl.ds(0, SUP)], rows[b], gsem[b]).wait()

        def fire_out(c, b):
            pltpu.async_copy(
                ybuf[b], out_hbm.at[pl.ds(c * SPC, SPC), :, wid], osem[b])

        def wait_out(b):
            pltpu.make_async_copy(
                ybuf[b], out_hbm.at[pl.ds(0, SPC), :, wid], osem[b]).wait()

        def compute(c, b):
            buf = rows[b]

            def group_body(gi, _):
                s_i = gi // 8
                b0 = (gi % 8) * 16
                rr = s_i * 128 + b0 + lanes
                pbase = (c * SPC + s_i) * HID
                par = plsc.load_gather(ibuf[b], [rr]) & 1
                par64 = par << 6

                zero = jnp.zeros((16,), jnp.float32)

                @plsc.parallel_loop(0, HID, 1, unroll=16,
                                    carry=(zero, zero))
                def _p1(k, carry):
                    s, s2 = carry
                    hvec = (lanes + k) & (HID - 1)
                    t = plsc.load_gather(buf, [rr, par64 + hvec])
                    p = plsc.load_gather(pos_v, [pbase + hvec])
                    x = t + p
                    plsc.store_scatter(xpack, [k * 16 + lanes], x)
                    return s + x, s2 + x * x

                s, s2 = _p1
                mean = s * (1.0 / HID)
                var = s2 * (1.0 / HID) - mean * mean
                rstd = _rsqrt(var + EPS)

                si_v = jnp.full((16,), s_i, jnp.int32)
                blv = b0 + lanes

                @plsc.parallel_loop(0, HID, 1, unroll=16)
                def _p2(h):
                    avec = (h - lanes) & (HID - 1)
                    xh = plsc.load_gather(xpack, [(avec << 4) + lanes])
                    gam = gamma_v[pl.ds(h * 16, 16)]
                    bet = beta_v[pl.ds(h * 16, 16)]
                    yv = (xh - mean) * rstd * gam + bet
                    plsc.store_scatter(
                        ybuf[b],
                        [si_v, jnp.full((16,), h >> 3, jnp.int32),
                         jnp.full((16,), h & 7, jnp.int32), blv], yv)
                return 0

            lax.fori_loop(0, GROUPS, group_body, 0)

        def body(c, b):
            wait_gather(b)

            @pl.when(c >= 2)
            def _():
                wait_out(b)

            @pl.when(c + 1 < NSUP)
            def _():
                fire_gather(c + 1, 1 - b)

            compute(c, b)
            fire_out(c, b)

        fire_gather(0, 0)

        def loop_body(c, _):
            @pl.when(c % 2 == 0)
            def _():
                body(c, 0)

            @pl.when(c % 2 == 1)
            def _():
                body(c, 1)
            return 0

        lax.fori_loop(0, NSUP, loop_body, 0)
        wait_out(0)
        wait_out(1)

    return emb


_emb_kernel = _make_emb_kernel()


def kernel(state, token_table, pos_table, ln_gamma, ln_beta):
    table2 = token_table.reshape(VOCAB // 2, 128)
    state_t = state.reshape(NW, BPW, SEQ).transpose(0, 2, 1).reshape(NW, RPW)
    pos_flat = pos_table.reshape(-1)
    gamma_splat = jnp.repeat(ln_gamma, 16)
    beta_splat = jnp.repeat(ln_beta, 16)
    out5 = _emb_kernel(state_t, table2, pos_flat, gamma_splat, beta_splat)
    return out5.transpose(2, 4, 0, 1, 3).reshape(BATCH, SEQ, HID)


# speedup vs baseline: 1.8795x; 1.0241x over previous
import functools

import jax
import jax.numpy as jnp
from jax import lax
from jax.experimental import pallas as pl
from jax.experimental.pallas import tpu as pltpu
from jax.experimental.pallas import tpu_sc as plsc

VOCAB = 1000000
SEQ = 200
HID = 64
BATCH = 4096
EPS = 1e-12

NW = 32
BPW = BATCH // NW
RPW = BPW * SEQ
SPC = 2
SUP = SPC * BPW
NSUP = SEQ // SPC
GROUPS = SUP // 16


def _rsqrt(v):
    i = plsc.bitcast(v, jnp.int32)
    i = jnp.int32(0x5F3759DF) - (i >> 1)
    y = plsc.bitcast(i, jnp.float32)
    for _ in range(3):
        y = y * (1.5 - 0.5 * v * y * y)
    return y


def _make_emb_kernel():
    mesh = plsc.VectorSubcoreMesh(core_axis_name="c", subcore_axis_name="s")

    @functools.partial(
        pl.kernel,
        mesh=mesh,
        compiler_params=pltpu.CompilerParams(
            needs_layout_passes=False, use_tc_tiling_on_sc=True),
        out_type=jax.ShapeDtypeStruct((SEQ, 8, NW, 8, 128), jnp.float32),
        scratch_types=[
            pltpu.VMEM((SUP, 128), jnp.float32),
            pltpu.VMEM((SUP, 128), jnp.float32),
            pltpu.VMEM((SPC, 8, 8, 128), jnp.float32),
            pltpu.VMEM((SPC, 8, 8, 128), jnp.float32),
            pltpu.VMEM((SUP,), jnp.int32),
            pltpu.VMEM((SUP,), jnp.int32),
            pltpu.VMEM((SUP,), jnp.int32),
            pltpu.VMEM((SUP,), jnp.int32),
            pltpu.VMEM((4 * HID * 16,), jnp.float32),
            pltpu.VMEM((SEQ * HID,), jnp.float32),
            pltpu.VMEM((HID * 16,), jnp.float32),
            pltpu.VMEM((HID * 16,), jnp.float32),
            pltpu.SemaphoreType.DMA,
            pltpu.SemaphoreType.DMA,
            pltpu.SemaphoreType.DMA,
            pltpu.SemaphoreType.DMA,
        ],
    )
    def emb(state_hbm, table_hbm, pos_hbm, gamma_hbm, beta_hbm, out_hbm,
            rows0, rows1, y0, y1, ib0, ib1, ih0, ih1,
            xpack, pos_v, gamma_v, beta_v, gs0, gs1, os0, os1):
        rows = (rows0, rows1)
        ybuf = (y0, y1)
        ibuf = (ib0, ib1)
        hbuf = (ih0, ih1)
        gsem = (gs0, gs1)
        osem = (os0, os1)
        wid = lax.axis_index("s") * 2 + lax.axis_index("c")
        pltpu.sync_copy(pos_hbm, pos_v)
        pltpu.sync_copy(gamma_hbm, gamma_v)
        pltpu.sync_copy(beta_hbm, beta_v)
        lanes = lax.iota(jnp.int32, 16)

        def fire_gather(c, b):
            pltpu.sync_copy(state_hbm.at[wid, pl.ds(c * SUP, SUP)], ibuf[b])
            for m in range(SUP // 16):
                iv = ibuf[b][pl.ds(m * 16, 16)]
                hbuf[b][pl.ds(m * 16, 16)] = iv >> 1
            for j in range(SPC):
                pltpu.async_copy(
                    table_hbm.at[hbuf[b].at[pl.ds(j * 128, 128)]],
                    rows[b].at[pl.ds(j * 128, 128)], gsem[b])

        def wait_gather(b):
            pltpu.make_async_copy(
                table_hbm.at[pl.ds(0, SUP)], rows[b], gsem[b]).wait()

        def fire_out(c, b):
            pltpu.async_copy(
                ybuf[b], out_hbm.at[pl.ds(c * SPC, SPC), :, wid], osem[b])

        def wait_out(b):
            pltpu.make_async_copy(
                ybuf[b], out_hbm.at[pl.ds(0, SPC), :, wid], osem[b]).wait()

        def compute(c, b):
            buf = rows[b]

            @plsc.parallel_loop(0, GROUPS, 1, unroll=2)
            def group_body(gi):
                s_i = gi // 8
                b0 = (gi % 8) * 16
                rr = s_i * 128 + b0 + lanes
                pbase = (c * SPC + s_i) * HID
                par = plsc.load_gather(ibuf[b], [rr]) & 1
                par64 = par << 6
                xbase = (gi & 3) * (HID * 16)

                zero = jnp.zeros((16,), jnp.float32)

                @plsc.parallel_loop(0, HID, 1, unroll=16,
                                    carry=(zero, zero))
                def _p1(k, carry):
                    s, s2 = carry
                    hvec = (lanes + k) & (HID - 1)
                    t = plsc.load_gather(buf, [rr, par64 + hvec])
                    p = plsc.load_gather(pos_v, [pbase + hvec])
                    x = t + p
                    plsc.store_scatter(xpack, [xbase + k * 16 + lanes], x)
                    return s + x, s2 + x * x

                s, s2 = _p1
                mean = s * (1.0 / HID)
                var = s2 * (1.0 / HID) - mean * mean
                rstd = _rsqrt(var + EPS)

                si_v = jnp.full((16,), s_i, jnp.int32)
                blv = b0 + lanes

                @plsc.parallel_loop(0, HID, 1, unroll=16)
                def _p2(h):
                    avec = (h - lanes) & (HID - 1)
                    xh = plsc.load_gather(xpack, [xbase + (avec << 4) + lanes])
                    gam = gamma_v[pl.ds(h * 16, 16)]
                    bet = beta_v[pl.ds(h * 16, 16)]
                    yv = (xh - mean) * rstd * gam + bet
                    plsc.store_scatter(
                        ybuf[b],
                        [si_v, jnp.full((16,), h >> 3, jnp.int32),
                         jnp.full((16,), h & 7, jnp.int32), blv], yv)

            del group_body

        def body(c, b):
            wait_gather(b)

            @pl.when(c >= 2)
            def _():
                wait_out(b)

            @pl.when(c + 1 < NSUP)
            def _():
                fire_gather(c + 1, 1 - b)

            compute(c, b)
            fire_out(c, b)

        fire_gather(0, 0)

        def loop_body(c, _):
            @pl.when(c % 2 == 0)
            def _():
                body(c, 0)

            @pl.when(c % 2 == 1)
            def _():
                body(c, 1)
            return 0

        lax.fori_loop(0, NSUP, loop_body, 0)
        wait_out(0)
        wait_out(1)

    return emb


_emb_kernel = _make_emb_kernel()


def kernel(state, token_table, pos_table, ln_gamma, ln_beta):
    table2 = token_table.reshape(VOCAB // 2, 128)
    state_t = state.reshape(NW, BPW, SEQ).transpose(0, 2, 1).reshape(NW, RPW)
    pos_flat = pos_table.reshape(-1)
    gamma_splat = jnp.repeat(ln_gamma, 16)
    beta_splat = jnp.repeat(ln_beta, 16)
    out5 = _emb_kernel(state_t, table2, pos_flat, gamma_splat, beta_splat)
    return out5.transpose(2, 4, 0, 1, 3).reshape(BATCH, SEQ, HID)


# speedup vs baseline: 1.9867x; 1.0570x over previous
import functools

import jax
import jax.numpy as jnp
from jax import lax
from jax.experimental import pallas as pl
from jax.experimental.pallas import tpu as pltpu
from jax.experimental.pallas import tpu_sc as plsc

VOCAB = 1000000
SEQ = 200
HID = 64
BATCH = 4096
EPS = 1e-12

NW = 32
BPW = BATCH // NW
RPW = BPW * SEQ
SPC = 2
SUP = SPC * BPW
NSUP = SEQ // SPC
GROUPS = SUP // 16


def _rsqrt(v):
    i = plsc.bitcast(v, jnp.int32)
    i = jnp.int32(0x5F3759DF) - (i >> 1)
    y = plsc.bitcast(i, jnp.float32)
    for _ in range(3):
        y = y * (1.5 - 0.5 * v * y * y)
    return y


def _make_emb_kernel():
    mesh = plsc.VectorSubcoreMesh(core_axis_name="c", subcore_axis_name="s")

    @functools.partial(
        pl.kernel,
        mesh=mesh,
        compiler_params=pltpu.CompilerParams(
            needs_layout_passes=False, use_tc_tiling_on_sc=True),
        out_type=jax.ShapeDtypeStruct((SEQ, 8, NW, 8, 128), jnp.float32),
        scratch_types=[
            pltpu.VMEM((SUP, 128), jnp.float32),
            pltpu.VMEM((SUP, 128), jnp.float32),
            pltpu.VMEM((SPC, 8, 8, 128), jnp.float32),
            pltpu.VMEM((SPC, 8, 8, 128), jnp.float32),
            pltpu.VMEM((SUP,), jnp.int32),
            pltpu.VMEM((SUP,), jnp.int32),
            pltpu.VMEM((SUP,), jnp.int32),
            pltpu.VMEM((SUP,), jnp.int32),
            pltpu.VMEM((SUP,), jnp.int32),
            pltpu.VMEM((SUP,), jnp.int32),
            pltpu.VMEM((4 * HID * 16,), jnp.float32),
            pltpu.VMEM((SEQ * HID,), jnp.float32),
            pltpu.VMEM((HID * 16,), jnp.float32),
            pltpu.VMEM((HID * 16,), jnp.float32),
            pltpu.SemaphoreType.DMA,
            pltpu.SemaphoreType.DMA,
            pltpu.SemaphoreType.DMA,
            pltpu.SemaphoreType.DMA,
            pltpu.SemaphoreType.DMA,
        ],
    )
    def emb(state_hbm, table_hbm, pos_hbm, gamma_hbm, beta_hbm, out_hbm,
            rows0, rows1, y0, y1, ib0, ib1, ih0, ih1, pb0, pb1,
            xpack, pos_v, gamma_v, beta_v, gs0, gs1, os0, os1, isem):
        rows = (rows0, rows1)
        ybuf = (y0, y1)
        ibuf = (ib0, ib1)
        hbuf = (ih0, ih1)
        pbuf = (pb0, pb1)
        gsem = (gs0, gs1)
        osem = (os0, os1)
        wid = lax.axis_index("s") * 2 + lax.axis_index("c")
        pltpu.sync_copy(pos_hbm, pos_v)
        pltpu.sync_copy(gamma_hbm, gamma_v)
        pltpu.sync_copy(beta_hbm, beta_v)
        lanes = lax.iota(jnp.int32, 16)

        def fire_idx(c, b):
            pltpu.async_copy(
                state_hbm.at[wid, pl.ds(c * SUP, SUP)], ibuf[b], isem)

        def wait_idx(b):
            pltpu.make_async_copy(
                state_hbm.at[wid, pl.ds(0, SUP)], ibuf[b], isem).wait()

        def fire_gather(c, b):
            for m in range(SUP // 16):
                iv = ibuf[b][pl.ds(m * 16, 16)]
                hbuf[b][pl.ds(m * 16, 16)] = iv >> 1
                pbuf[b][pl.ds(m * 16, 16)] = iv & 1
            for j in range(SPC):
                pltpu.async_copy(
                    table_hbm.at[hbuf[b].at[pl.ds(j * 128, 128)]],
                    rows[b].at[pl.ds(j * 128, 128)], gsem[b])

        def wait_gather(b):
            pltpu.make_async_copy(
                table_hbm.at[pl.ds(0, SUP)], rows[b], gsem[b]).wait()

        def fire_out(c, b):
            pltpu.async_copy(
                ybuf[b], out_hbm.at[pl.ds(c * SPC, SPC), :, wid], osem[b])

        def wait_out(b):
            pltpu.make_async_copy(
                ybuf[b], out_hbm.at[pl.ds(0, SPC), :, wid], osem[b]).wait()

        def compute(c, b):
            buf = rows[b]

            @plsc.parallel_loop(0, GROUPS, 1, unroll=2)
            def group_body(gi):
                s_i = gi // 8
                b0 = (gi % 8) * 16
                rr = s_i * 128 + b0 + lanes
                pbase = (c * SPC + s_i) * HID
                par = plsc.load_gather(pbuf[b], [rr])
                par64 = par << 6
                xbase = (gi & 3) * (HID * 16)

                zero = jnp.zeros((16,), jnp.float32)

                @plsc.parallel_loop(0, HID, 1, unroll=16,
                                    carry=(zero, zero))
                def _p1(k, carry):
                    s, s2 = carry
                    hvec = (lanes + k) & (HID - 1)
                    t = plsc.load_gather(buf, [rr, par64 + hvec])
                    p = plsc.load_gather(pos_v, [pbase + hvec])
                    x = t + p
                    plsc.store_scatter(xpack, [xbase + k * 16 + lanes], x)
                    return s + x, s2 + x * x

                s, s2 = _p1
                mean = s * (1.0 / HID)
                var = s2 * (1.0 / HID) - mean * mean
                rstd = _rsqrt(var + EPS)

                si_v = jnp.full((16,), s_i, jnp.int32)
                blv = b0 + lanes

                @plsc.parallel_loop(0, HID, 1, unroll=16)
                def _p2(h):
                    avec = (h - lanes) & (HID - 1)
                    xh = plsc.load_gather(xpack, [xbase + (avec << 4) + lanes])
                    gam = gamma_v[pl.ds(h * 16, 16)]
                    bet = beta_v[pl.ds(h * 16, 16)]
                    yv = (xh - mean) * rstd * gam + bet
                    plsc.store_scatter(
                        ybuf[b],
                        [si_v, jnp.full((16,), h >> 3, jnp.int32),
                         jnp.full((16,), h & 7, jnp.int32), blv], yv)

            del group_body

        def body(c, b):
            wait_gather(b)

            @pl.when(c >= 2)
            def _():
                wait_out(b)

            @pl.when(c + 1 < NSUP)
            def _():
                wait_idx(1 - b)
                fire_gather(c + 1, 1 - b)

                @pl.when(c + 2 < NSUP)
                def _():
                    fire_idx(c + 2, b)

            compute(c, b)
            fire_out(c, b)

        fire_idx(0, 0)
        wait_idx(0)
        fire_gather(0, 0)
        fire_idx(1, 1)

        def loop_body(c, _):
            @pl.when(c % 2 == 0)
            def _():
                body(c, 0)

            @pl.when(c % 2 == 1)
            def _():
                body(c, 1)
            return 0

        lax.fori_loop(0, NSUP, loop_body, 0)
        wait_out(0)
        wait_out(1)

    return emb


_emb_kernel = _make_emb_kernel()


def kernel(state, token_table, pos_table, ln_gamma, ln_beta):
    table2 = token_table.reshape(VOCAB // 2, 128)
    state_t = state.reshape(NW, BPW, SEQ).transpose(0, 2, 1).reshape(NW, RPW)
    pos_flat = pos_table.reshape(-1)
    gamma_splat = jnp.repeat(ln_gamma, 16)
    beta_splat = jnp.repeat(ln_beta, 16)
    out5 = _emb_kernel(state_t, table2, pos_flat, gamma_splat, beta_splat)
    return out5.transpose(2, 4, 0, 1, 3).reshape(BATCH, SEQ, HID)
